# baseline (speedup 1.0000x reference)
import jax
import jax.numpy as jnp
from jax.experimental import pallas as pl

_S = 16
_H = 50
_W = 50
_N = _H * _W
_K = 768
_O = 512
_BLK_N = 512


def _mm_kernel(w_ref, p_ref, b_ref, o_ref):
    o_ref[...] = (
        jax.lax.dot_general(
            w_ref[...], p_ref[...],
            (((1,), (1,)), ((), ())),
            preferred_element_type=jnp.float32,
        )
        + b_ref[...]
    )


def kernel(x, W_ext, b_ext, W_conv1, b_conv1, W_loc, b_loc, W_score, b_score):
    x5 = x.astype(jnp.bfloat16).reshape(3, _H, _S, _W, _S)
    patches = jax.lax.reshape(x5, (_N, _K), dimensions=(1, 3, 0, 2, 4))
    w_flat = W_ext.reshape(_O, _K).astype(jnp.bfloat16)
    bias = b_ext.reshape(_O, 1)

    out = pl.pallas_call(
        _mm_kernel,
        grid=(pl.cdiv(_N, _BLK_N),),
        in_specs=[
            pl.BlockSpec((_O, _K), lambda n: (0, 0)),
            pl.BlockSpec((_BLK_N, _K), lambda n: (n, 0)),
            pl.BlockSpec((_O, 1), lambda n: (0, 0)),
        ],
        out_specs=pl.BlockSpec((_O, _BLK_N), lambda n: (0, n)),
        out_shape=jax.ShapeDtypeStruct((_O, _N), jnp.float32),
    )(w_flat, patches, bias)

    return out.reshape(1, _O, _H, _W)


# speedup vs baseline: 1.9323x; 1.9323x over previous
import jax
import jax.numpy as jnp
from jax.experimental import pallas as pl

_S = 16
_H = 50
_W = 50
_K = 768
_O = 512
_G = 8


_LW = 128


def _conv_kernel(x_ref, s2_ref, w_ref, b_ref, o_ref):
    X = x_ref[0].astype(jnp.bfloat16)
    X2 = X.reshape(3 * _G * _S, _W * _S)
    X3 = jnp.dot(X2, s2_ref[...],
                 preferred_element_type=jnp.float32).astype(jnp.bfloat16)
    cols = []
    for ii in range(_G):
        X3i = jnp.concatenate(
            [X3[c * _G * _S + ii * _S: c * _G * _S + ii * _S + _S, :]
             for c in range(3)], axis=0)
        cols.append(jnp.concatenate(
            [X3i[:, dx * _LW:(dx + 1) * _LW] for dx in range(_S)],
            axis=0))
    P = jnp.concatenate(cols, axis=1)
    Y = jnp.dot(w_ref[...], P, preferred_element_type=jnp.float32) + b_ref[...]
    for ii in range(_G):
        o_ref[:, ii, :] = Y[:, ii * _LW: ii * _LW + _W]


def kernel(x, W_ext, b_ext, W_conv1, b_conv1, W_loc, b_loc, W_score, b_score):
    m = jnp.arange(_W * _S)
    s2 = jax.nn.one_hot((m % _S) * _LW + m // _S, _S * _LW, dtype=jnp.bfloat16)
    w_perm = (W_ext.transpose(0, 3, 1, 2)
              .reshape(_O, _K).astype(jnp.bfloat16))
    bias = b_ext.reshape(_O, 1)

    out = pl.pallas_call(
        _conv_kernel,
        grid=(pl.cdiv(_H, _G),),
        in_specs=[
            pl.BlockSpec((1, 3, _G * _S, _W * _S), lambda n: (0, 0, n, 0)),
            pl.BlockSpec((_W * _S, _S * _LW), lambda n: (0, 0)),
            pl.BlockSpec((_O, _K), lambda n: (0, 0)),
            pl.BlockSpec((_O, 1), lambda n: (0, 0)),
        ],
        out_specs=pl.BlockSpec((_O, _G, _W), lambda n: (0, n, 0)),
        out_shape=jax.ShapeDtypeStruct((_O, _H, _W), jnp.float32),
    )(x, s2, w_perm, bias)

    return out.reshape(1, _O, _H, _W)


# speedup vs baseline: 1.9921x; 1.0310x over previous
import numpy as np
import jax
import jax.numpy as jnp
from jax.experimental import pallas as pl
from jax.experimental.pallas import tpu as pltpu

_S = 16
_H = 50
_W = 50
_K = 768
_O = 512
_G = 8
_LW = 128

_S2_NP = np.zeros((_W * _S, _S * _LW), dtype=np.float32)
_m = np.arange(_W * _S)
_S2_NP[_m, (_m % _S) * _LW + _m // _S] = 1.0

_S3_NP = np.zeros((_K, _K), dtype=np.float32)
_k = np.arange(_K)
_c, _dy, _dx = _k // (_S * _S), (_k // _S) % _S, _k % _S
_S3_NP[_k, _dx * 48 + _c * _S + _dy] = 1.0


def _conv_kernel(x_ref, s2_ref, s3_ref, w_ref, b_ref, o_ref, wp_ref):
    @pl.when(pl.program_id(0) == 0)
    def _permute_weights():
        wp_ref[...] = jnp.dot(
            w_ref[...], s3_ref[...], preferred_element_type=jnp.float32
        ).astype(jnp.bfloat16)

    X = x_ref[0].astype(jnp.bfloat16)
    X2 = X.reshape(3 * _G * _S, _W * _S)
    X3 = jnp.dot(X2, s2_ref[...],
                 preferred_element_type=jnp.float32).astype(jnp.bfloat16)
    cols = []
    for ii in range(_G):
        X3i = jnp.concatenate(
            [X3[c * _G * _S + ii * _S: c * _G * _S + ii * _S + _S, :]
             for c in range(3)], axis=0)
        cols.append(jnp.concatenate(
            [X3i[:, dx * _LW:(dx + 1) * _LW] for dx in range(_S)],
            axis=0))
    P = jnp.concatenate(cols, axis=1)
    Y = jnp.dot(wp_ref[...], P, preferred_element_type=jnp.float32) + b_ref[...]
    for ii in range(_G):
        o_ref[:, ii, :] = Y[:, ii * _LW: ii * _LW + _W]


def kernel(x, W_ext, b_ext, W_conv1, b_conv1, W_loc, b_loc, W_score, b_score):
    s2 = jnp.asarray(_S2_NP, dtype=jnp.bfloat16)
    s3 = jnp.asarray(_S3_NP, dtype=jnp.bfloat16)
    w_flat = W_ext.reshape(_O, _K).astype(jnp.bfloat16)
    bias = b_ext.reshape(_O, 1)

    out = pl.pallas_call(
        _conv_kernel,
        grid=(pl.cdiv(_H, _G),),
        in_specs=[
            pl.BlockSpec((1, 3, _G * _S, _W * _S), lambda n: (0, 0, n, 0)),
            pl.BlockSpec((_W * _S, _S * _LW), lambda n: (0, 0)),
            pl.BlockSpec((_K, _K), lambda n: (0, 0)),
            pl.BlockSpec((_O, _K), lambda n: (0, 0)),
            pl.BlockSpec((_O, 1), lambda n: (0, 0)),
        ],
        out_specs=pl.BlockSpec((_O, _G, _W), lambda n: (0, n, 0)),
        out_shape=jax.ShapeDtypeStruct((_O, _H, _W), jnp.float32),
        scratch_shapes=[pltpu.VMEM((_O, _K), jnp.bfloat16)],
    )(x, s2, s3, w_flat, bias)

    return out.reshape(1, _O, _H, _W)


# speedup vs baseline: 3.2628x; 1.6379x over previous
import numpy as np
import jax
import jax.numpy as jnp
from jax.experimental import pallas as pl
from jax.experimental.pallas import tpu as pltpu

_S = 16
_H = 50
_W = 50
_K = 768
_O = 512
_G = 8
_LW = 128

_S2_NP = np.zeros((_W * _S, _S * _LW), dtype=np.float32)
_m = np.arange(_W * _S)
_S2_NP[_m, (_m % _S) * _LW + _m // _S] = 1.0

_S3T_NP = np.zeros((_K, _K), dtype=np.float32)
_k = np.arange(_K)
_c, _dy, _dx = _k // (_S * _S), (_k // _S) % _S, _k % _S
_S3T_NP[_dx * 48 + _c * _S + _dy, _k] = 1.0


def _conv_kernel(x_ref, s2_ref, s3t_ref, wt_ref, b_ref, o_ref, wp_ref):
    @pl.when(pl.program_id(0) == 0)
    def _permute_weights():
        wp_ref[...] = jnp.dot(
            s3t_ref[...], wt_ref[...], preferred_element_type=jnp.float32
        ).astype(jnp.bfloat16)

    X = x_ref[0].astype(jnp.bfloat16)
    X2 = X.reshape(3 * _G * _S, _W * _S)
    X3 = jnp.dot(X2, s2_ref[...],
                 preferred_element_type=jnp.float32).astype(jnp.bfloat16)
    cols = []
    for ii in range(_G):
        X3i = jnp.concatenate(
            [X3[c * _G * _S + ii * _S: c * _G * _S + ii * _S + _S, :]
             for c in range(3)], axis=0)
        cols.append(jnp.concatenate(
            [X3i[:, dx * _LW:(dx + 1) * _LW] for dx in range(_S)],
            axis=0))
    P = jnp.concatenate(cols, axis=1)
    Y = jax.lax.dot_general(
        P, wp_ref[...], (((0,), (0,)), ((), ())),
        preferred_element_type=jnp.float32,
    ) + b_ref[...]
    for ii in range(_G):
        o_ref[ii, :, :] = Y[ii * _LW: ii * _LW + _W, :]


def kernel(x, W_ext, b_ext, W_conv1, b_conv1, W_loc, b_loc, W_score, b_score):
    s2 = jnp.asarray(_S2_NP, dtype=jnp.bfloat16)
    s3t = jnp.asarray(_S3T_NP, dtype=jnp.bfloat16)
    w_t = W_ext.reshape(_O, _K).T.astype(jnp.bfloat16)
    bias = b_ext.reshape(1, _O)

    out = pl.pallas_call(
        _conv_kernel,
        grid=(pl.cdiv(_H, _G),),
        in_specs=[
            pl.BlockSpec((1, 3, _G * _S, _W * _S), lambda n: (0, 0, n, 0)),
            pl.BlockSpec((_W * _S, _S * _LW), lambda n: (0, 0)),
            pl.BlockSpec((_K, _K), lambda n: (0, 0)),
            pl.BlockSpec((_K, _O), lambda n: (0, 0)),
            pl.BlockSpec((1, _O), lambda n: (0, 0)),
        ],
        out_specs=pl.BlockSpec((_G, _W, _O), lambda n: (n, 0, 0)),
        out_shape=jax.ShapeDtypeStruct((_H, _W, _O), jnp.float32),
        scratch_shapes=[pltpu.VMEM((_K, _O), jnp.bfloat16)],
    )(x, s2, s3t, w_t, bias)

    return out.transpose(2, 0, 1).reshape(1, _O, _H, _W)


# speedup vs baseline: 4.3189x; 1.3237x over previous
import numpy as np
import jax
import jax.numpy as jnp
from jax.experimental import pallas as pl
from jax.experimental.pallas import tpu as pltpu

_S = 16
_H = 50
_W = 50
_K = 768
_O = 512
_G = 8
_LW = 64

_S2_NP = np.zeros((_W * _S, _S * _LW), dtype=np.float32)
_m = np.arange(_W * _S)
_S2_NP[_m, (_m % _S) * _LW + _m // _S] = 1.0

_S3T_NP = np.zeros((_K, _K), dtype=np.float32)
_k = np.arange(_K)
_c, _dy, _dx = _k // (_S * _S), (_k // _S) % _S, _k % _S
_S3T_NP[_dx * 48 + _c * _S + _dy, _k] = 1.0


def _conv_kernel(x_ref, s2_ref, s3t_ref, wt_ref, b_ref, o_ref, wp_ref):
    @pl.when(pl.program_id(0) == 0)
    def _permute_weights():
        wp_ref[...] = jnp.dot(
            s3t_ref[...], wt_ref[...].astype(jnp.bfloat16),
            preferred_element_type=jnp.float32,
        ).astype(jnp.bfloat16)

    X = x_ref[0].astype(jnp.bfloat16)
    X2 = X.reshape(3 * _G * _S, _W * _S)
    X3 = jnp.dot(X2, s2_ref[...],
                 preferred_element_type=jnp.float32).astype(jnp.bfloat16)
    cols = []
    for ii in range(_G):
        X3i = jnp.concatenate(
            [X3[c * _G * _S + ii * _S: c * _G * _S + ii * _S + _S, :]
             for c in range(3)], axis=0)
        cols.append(jnp.concatenate(
            [X3i[:, dx * _LW:(dx + 1) * _LW] for dx in range(_S)],
            axis=0))
    P = jnp.concatenate(cols, axis=1)
    Y = jax.lax.dot_general(
        P, wp_ref[...], (((0,), (0,)), ((), ())),
        preferred_element_type=jnp.float32,
    ) + b_ref[...]
    for ii in range(_G):
        o_ref[ii, :, :] = Y[ii * _LW: ii * _LW + _W, :]


def kernel(x, W_ext, b_ext, W_conv1, b_conv1, W_loc, b_loc, W_score, b_score):
    s2 = jnp.asarray(_S2_NP, dtype=jnp.bfloat16)
    s3t = jnp.asarray(_S3T_NP, dtype=jnp.bfloat16)
    w_t = W_ext.reshape(_O, _K).T
    bias = b_ext.reshape(1, _O)

    out = pl.pallas_call(
        _conv_kernel,
        grid=(pl.cdiv(_H, _G),),
        in_specs=[
            pl.BlockSpec((1, 3, _G * _S, _W * _S), lambda n: (0, 0, n, 0)),
            pl.BlockSpec((_W * _S, _S * _LW), lambda n: (0, 0)),
            pl.BlockSpec((_K, _K), lambda n: (0, 0)),
            pl.BlockSpec((_K, _O), lambda n: (0, 0)),
            pl.BlockSpec((1, _O), lambda n: (0, 0)),
        ],
        out_specs=pl.BlockSpec((_G, _W, _O), lambda n: (n, 0, 0)),
        out_shape=jax.ShapeDtypeStruct((_H, _W, _O), jnp.float32),
        scratch_shapes=[pltpu.VMEM((_K, _O), jnp.bfloat16)],
    )(x, s2, s3t, w_t, bias)

    return out.transpose(2, 0, 1).reshape(1, _O, _H, _W)


# speedup vs baseline: 4.6601x; 1.0790x over previous
import numpy as np
import jax
import jax.numpy as jnp
from jax.experimental import pallas as pl
from jax.experimental.pallas import tpu as pltpu

_S = 16
_H = 50
_W = 50
_K = 768
_O = 512
_G = 10
_LW = 64

_S2_NP = np.zeros((_W * _S, _S * _LW), dtype=np.float32)
_m = np.arange(_W * _S)
_S2_NP[_m, (_m % _S) * _LW + _m // _S] = 1.0

_S3T_NP = np.zeros((_K, _K), dtype=np.float32)
_k = np.arange(_K)
_c, _dy, _dx = _k // (_S * _S), (_k // _S) % _S, _k % _S
_S3T_NP[_dx * 48 + _c * _S + _dy, _k] = 1.0


def _conv_kernel(x_ref, s2_ref, s3t_ref, wt_ref, b_ref, o_ref, wp_ref):
    @pl.when(pl.program_id(0) == 0)
    def _permute_weights():
        wp_ref[...] = jnp.dot(
            s3t_ref[...], wt_ref[...].astype(jnp.bfloat16),
            preferred_element_type=jnp.float32,
        ).astype(jnp.bfloat16)

    X = x_ref[0].astype(jnp.bfloat16)
    X2 = X.reshape(3 * _G * _S, _W * _S)
    X3 = jnp.dot(X2, s2_ref[...],
                 preferred_element_type=jnp.float32).astype(jnp.bfloat16)
    cols = []
    for ii in range(_G):
        X3i = jnp.concatenate(
            [X3[c * _G * _S + ii * _S: c * _G * _S + ii * _S + _S, :]
             for c in range(3)], axis=0)
        cols.append(jnp.concatenate(
            [X3i[:, dx * _LW:(dx + 1) * _LW] for dx in range(_S)],
            axis=0))
    P = jnp.concatenate(cols, axis=1)
    Y = jax.lax.dot_general(
        P, wp_ref[...], (((0,), (0,)), ((), ())),
        preferred_element_type=jnp.float32,
    ) + b_ref[...]
    for ii in range(_G):
        o_ref[ii, :, :] = Y[ii * _LW: ii * _LW + _W, :]


def kernel(x, W_ext, b_ext, W_conv1, b_conv1, W_loc, b_loc, W_score, b_score):
    s2 = jnp.asarray(_S2_NP, dtype=jnp.bfloat16)
    s3t = jnp.asarray(_S3T_NP, dtype=jnp.bfloat16)
    w_t = W_ext.reshape(_O, _K).T
    bias = b_ext.reshape(1, _O)

    out = pl.pallas_call(
        _conv_kernel,
        grid=(pl.cdiv(_H, _G),),
        in_specs=[
            pl.BlockSpec((1, 3, _G * _S, _W * _S), lambda n: (0, 0, n, 0)),
            pl.BlockSpec((_W * _S, _S * _LW), lambda n: (0, 0)),
            pl.BlockSpec((_K, _K), lambda n: (0, 0)),
            pl.BlockSpec((_K, _O), lambda n: (0, 0)),
            pl.BlockSpec((1, _O), lambda n: (0, 0)),
        ],
        out_specs=pl.BlockSpec((_G, _W, _O), lambda n: (n, 0, 0)),
        out_shape=jax.ShapeDtypeStruct((_H, _W, _O), jnp.float32),
        scratch_shapes=[pltpu.VMEM((_K, _O), jnp.bfloat16)],
    )(x, s2, s3t, w_t, bias)

    return out.transpose(2, 0, 1).reshape(1, _O, _H, _W)
